# baseline (speedup 1.0000x reference)
import functools

import jax
import jax.numpy as jnp
import numpy as np
from jax import lax
from jax.experimental import pallas as pl
from jax.experimental.pallas import tpu as pltpu
from jax.experimental.pallas import tpu_sc as plsc

_STRENGTH = 0.1
_NEG_INF = float("-inf")
_INT_MIN = np.int32(-(2 ** 31))
_INT_MAX = np.int32(2 ** 31 - 1)
_LOW_MASK = np.int32(16383)
_HIGH_MASK = np.int32(-16384)

_NC = 2
_NS = 16
_LANES = 16
_KPAD = 16
_CHUNK = 8


def _proj_norm_kernel(feat_ref, wt_ref, proj_ref, hi_ref, lo_ref):
    proj = jnp.dot(feat_ref[...], wt_ref[...],
                   preferred_element_type=jnp.float32,
                   precision=jax.lax.Precision.HIGHEST)
    proj_ref[...] = proj
    norm = jnp.sqrt(jnp.sum(proj * proj, axis=1, keepdims=True))
    normed = proj / jnp.maximum(norm, 1e-12)
    hi = normed.astype(jnp.bfloat16)
    hi_ref[...] = hi
    lo_ref[...] = (normed - hi.astype(jnp.float32)).astype(jnp.bfloat16)


def _f32_to_ikey(x):
    bits = jax.lax.bitcast_convert_type(x, jnp.int32)
    return bits ^ (jax.lax.shift_right_arithmetic(bits, 31) & _INT_MAX)


def _ikey_to_f32(k):
    bits = k ^ (jax.lax.shift_right_arithmetic(k, 31) & _INT_MAX)
    return jax.lax.bitcast_convert_type(bits, jnp.float32)


def _topk_kernel(rhi_ref, rlo_ref, thi_ref, tlo_ref, w_ref, id_ref, keys_ref,
                 *, block_rows, n, kp1):
    pid = pl.program_id(0)
    row0 = pid * block_rows

    dot = functools.partial(jnp.dot, preferred_element_type=jnp.float32,
                            precision=jax.lax.Precision.DEFAULT)
    sims = (dot(rhi_ref[...], thi_ref[...])
            + dot(rhi_ref[...], tlo_ref[...])
            + dot(rlo_ref[...], thi_ref[...]))
    col_iota = jax.lax.broadcasted_iota(jnp.int32, (block_rows, n), 1)
    keys_ref[...] = (_f32_to_ikey(sims) & _HIGH_MASK) | (_LOW_MASK - col_iota)

    vals = []
    idxs = []
    m_prev = jnp.full((block_rows, 1), _INT_MAX, jnp.int32)
    for _ in range(kp1):
        cand = jnp.where(keys_ref[...] < m_prev, keys_ref[...], _INT_MIN)
        m = jnp.max(cand, axis=1, keepdims=True)
        idxs.append(_LOW_MASK - (m & _LOW_MASK))
        vals.append(_ikey_to_f32(m & _HIGH_MASK))
        m_prev = m

    row_ids = row0 + jax.lax.broadcasted_iota(jnp.int32, (block_rows, 1), 0)

    valid = [i != row_ids for i in idxs]
    mmax = functools.reduce(
        jnp.maximum,
        [jnp.where(v, x, _NEG_INF) for v, x in zip(valid, vals)])
    exps = [jnp.where(v, jnp.exp(x - mmax), 0.0)
            for v, x in zip(valid, vals)]
    denom = functools.reduce(jnp.add, exps)
    weights = [e / denom for e in exps]
    ids = [jnp.where(v, i, 0) for v, i in zip(valid, idxs)]

    zero_i = jnp.zeros((block_rows, 1), jnp.int32)
    id_ref[...] = jnp.concatenate(
        ids + [zero_i] * (_KPAD - kp1), axis=1)
    zero_wb = jnp.zeros((block_rows, 1, _LANES), jnp.float32)
    wb = jnp.concatenate(
        [jnp.broadcast_to(w.reshape(block_rows, 1, 1),
                          (block_rows, 1, _LANES)) for w in weights]
        + [zero_wb] * (_KPAD - kp1), axis=1)
    w_ref[...] = wb.reshape(block_rows * _KPAD, _LANES)


def _sc_gather_kernel(proj_hbm, idx_hbm, w_hbm, feat_hbm, emb_hbm, out_hbm,
                      idx_v0, idx_v1, w_v0, w_v1, rows_v0, rows_v1,
                      feat_v0, feat_v1, out_v0, out_v1, emb_v,
                      sem0, sem1, *, rows_per_worker):
    wid = lax.axis_index("s") * _NC + lax.axis_index("c")
    npairs = rows_per_worker // (2 * _CHUNK)
    nd = 128 // _LANES
    pltpu.sync_copy(emb_hbm, emb_v)
    embs = [emb_v[pl.ds(dd * _LANES, _LANES)] for dd in range(nd)]
    bufs = [
        (idx_v0, w_v0, rows_v0, feat_v0, out_v0, sem0),
        (idx_v1, w_v1, rows_v1, feat_v1, out_v1, sem1),
    ]

    def pair_body(p, _):
        base = wid * rows_per_worker + p * 2 * _CHUNK
        gathers = []
        for b, (idx_v, w_v, rows_v, feat_v, out_v, sem) in enumerate(bufs):
            row0 = base + b * _CHUNK
            flat0 = row0 * _KPAD
            pltpu.sync_copy(idx_hbm.at[pl.ds(flat0, _CHUNK * _KPAD)], idx_v)
            gathers.append(pltpu.async_copy(proj_hbm.at[idx_v], rows_v, sem))
            pltpu.sync_copy(w_hbm.at[pl.ds(flat0, _CHUNK * _KPAD)], w_v)
            pltpu.sync_copy(feat_hbm.at[pl.ds(row0, _CHUNK)], feat_v)
        for b, (idx_v, w_v, rows_v, feat_v, out_v, sem) in enumerate(bufs):
            row0 = base + b * _CHUNK
            gathers[b].wait()
            for r in range(_CHUNK):
                wks = [w_v[r * _KPAD + k, :] for k in range(_KPAD)]
                for d in range(nd):
                    dsl = pl.ds(d * _LANES, _LANES)
                    acc0 = wks[0] * rows_v[r * _KPAD, dsl]
                    acc1 = wks[1] * rows_v[r * _KPAD + 1, dsl]
                    for k in range(2, _KPAD, 2):
                        acc0 = acc0 + wks[k] * rows_v[r * _KPAD + k, dsl]
                        acc1 = acc1 + wks[k + 1] * rows_v[r * _KPAD + k + 1, dsl]
                    out_v[r, dsl] = (feat_v[r, dsl]
                                     + _STRENGTH * (acc0 + acc1 + embs[d]))
            pltpu.sync_copy(out_v, out_hbm.at[pl.ds(row0, _CHUNK)])
        return 0

    lax.fori_loop(0, npairs, pair_body, 0)


def kernel(feats, W, emb):
    feat = feats[0]
    n, d = feat.shape
    k = min(10, n // 10)
    kp1 = k + 1

    block_rows = 400 if n % 400 == 0 else 200

    proj, normed_hi, normed_lo = pl.pallas_call(
        _proj_norm_kernel,
        grid=(n // block_rows,),
        in_specs=[
            pl.BlockSpec((block_rows, d), lambda i: (i, 0)),
            pl.BlockSpec((d, d), lambda i: (0, 0)),
        ],
        out_specs=[
            pl.BlockSpec((block_rows, d), lambda i: (i, 0)),
            pl.BlockSpec((block_rows, d), lambda i: (i, 0)),
            pl.BlockSpec((block_rows, d), lambda i: (i, 0)),
        ],
        out_shape=[
            jax.ShapeDtypeStruct((n, d), jnp.float32),
            jax.ShapeDtypeStruct((n, d), jnp.bfloat16),
            jax.ShapeDtypeStruct((n, d), jnp.bfloat16),
        ],
    )(feat, W.T)

    w16, id16 = pl.pallas_call(
        functools.partial(_topk_kernel, block_rows=block_rows, n=n, kp1=kp1),
        grid=(n // block_rows,),
        in_specs=[
            pl.BlockSpec((block_rows, d), lambda i: (i, 0)),
            pl.BlockSpec((block_rows, d), lambda i: (i, 0)),
            pl.BlockSpec((d, n), lambda i: (0, 0)),
            pl.BlockSpec((d, n), lambda i: (0, 0)),
        ],
        out_specs=[
            pl.BlockSpec((block_rows * _KPAD, _LANES), lambda i: (i, 0)),
            pl.BlockSpec((block_rows, _KPAD), lambda i: (i, 0)),
        ],
        out_shape=[
            jax.ShapeDtypeStruct((n * _KPAD, _LANES), jnp.float32),
            jax.ShapeDtypeStruct((n, _KPAD), jnp.int32),
        ],
        scratch_shapes=[pltpu.VMEM((block_rows, n), jnp.int32)],
    )(normed_hi, normed_lo, normed_hi.T, normed_lo.T)

    nw = _NC * _NS
    rows_per_worker = -(-n // (nw * _CHUNK)) * _CHUNK
    npad = rows_per_worker * nw
    pad = npad - n
    idx_flat = jnp.pad(id16, ((0, pad), (0, 0))).reshape(-1)
    w_bcast = jnp.pad(w16, ((0, pad * _KPAD), (0, 0)))
    feat_pad = jnp.pad(feat, ((0, pad), (0, 0)))

    mesh = plsc.VectorSubcoreMesh(core_axis_name="c", subcore_axis_name="s")
    sc = pl.kernel(
        functools.partial(_sc_gather_kernel, rows_per_worker=rows_per_worker),
        mesh=mesh,
        out_type=jax.ShapeDtypeStruct((npad, d), jnp.float32),
        scratch_types=[
            pltpu.VMEM((_CHUNK * _KPAD,), jnp.int32),
            pltpu.VMEM((_CHUNK * _KPAD,), jnp.int32),
            pltpu.VMEM((_CHUNK * _KPAD, _LANES), jnp.float32),
            pltpu.VMEM((_CHUNK * _KPAD, _LANES), jnp.float32),
            pltpu.VMEM((_CHUNK * _KPAD, d), jnp.float32),
            pltpu.VMEM((_CHUNK * _KPAD, d), jnp.float32),
            pltpu.VMEM((_CHUNK, d), jnp.float32),
            pltpu.VMEM((_CHUNK, d), jnp.float32),
            pltpu.VMEM((_CHUNK, d), jnp.float32),
            pltpu.VMEM((_CHUNK, d), jnp.float32),
            pltpu.VMEM((d,), jnp.float32),
            pltpu.SemaphoreType.DMA,
            pltpu.SemaphoreType.DMA,
        ],
    )
    out = sc(proj, idx_flat, w_bcast, feat_pad, emb.reshape(-1))

    return out[:n][None]


# speedup vs baseline: 2.3676x; 2.3676x over previous
import functools

import jax
import jax.numpy as jnp
import numpy as np
from jax.experimental import pallas as pl
from jax.experimental.pallas import tpu as pltpu

_STRENGTH = 0.1
_NEG_INF = float("-inf")
_INT_MIN = np.int32(-(2 ** 31))
_INT_MAX = np.int32(2 ** 31 - 1)
_LOW_MASK = np.int32(16383)
_HIGH_MASK = np.int32(-16384)


def _proj_norm_kernel(feat_ref, wt_ref, proj_ref, hi_ref, lo_ref):
    proj = jnp.dot(feat_ref[...], wt_ref[...],
                   preferred_element_type=jnp.float32,
                   precision=jax.lax.Precision.HIGHEST)
    proj_ref[...] = proj
    norm = jnp.sqrt(jnp.sum(proj * proj, axis=1, keepdims=True))
    normed = proj / jnp.maximum(norm, 1e-12)
    hi = normed.astype(jnp.bfloat16)
    hi_ref[...] = hi
    lo_ref[...] = (normed - hi.astype(jnp.float32)).astype(jnp.bfloat16)


def _f32_to_ikey(x):
    bits = jax.lax.bitcast_convert_type(x, jnp.int32)
    return bits ^ (jax.lax.shift_right_arithmetic(bits, 31) & _INT_MAX)


def _ikey_to_f32(k):
    bits = k ^ (jax.lax.shift_right_arithmetic(k, 31) & _INT_MAX)
    return jax.lax.bitcast_convert_type(bits, jnp.float32)


def _topk_prop_kernel(rhi_ref, rlo_ref, thi_ref, tlo_ref, proj_ref, feat_ref,
                      emb_ref, out_ref, keys_ref, *, block_rows, n, npk, kp1):
    pid = pl.program_id(0)
    row0 = pid * block_rows

    dot = functools.partial(jnp.dot, preferred_element_type=jnp.float32,
                            precision=jax.lax.Precision.DEFAULT)
    sims = (dot(rhi_ref[...], thi_ref[...])
            + dot(rhi_ref[...], tlo_ref[...])
            + dot(rlo_ref[...], thi_ref[...]))
    col_iota = jax.lax.broadcasted_iota(jnp.int32, (block_rows, n), 1)
    keys = (_f32_to_ikey(sims) & _HIGH_MASK) | (_LOW_MASK - col_iota)
    if npk > n:
        keys = jnp.concatenate(
            [keys, jnp.full((block_rows, npk - n), _INT_MIN, jnp.int32)],
            axis=1)
    keys_ref[...] = keys

    ms = []
    m_prev = jnp.full((block_rows, 1), _INT_MAX, jnp.int32)
    for _ in range((kp1 + 1) // 2):
        x = jnp.where(keys_ref[...] < m_prev, keys_ref[...], _INT_MIN)
        a = x
        b = None
        w = npk
        while w > 1000 and w % 256 == 0:
            h = w // 2
            aL, aR = a[:, :h], a[:, h:]
            if b is None:
                b = jnp.minimum(aL, aR)
            else:
                b = jnp.maximum(jnp.minimum(aL, aR),
                                jnp.maximum(b[:, :h], b[:, h:]))
            a = jnp.maximum(aL, aR)
            w = h
        m1 = jnp.max(a, axis=1, keepdims=True)
        sec_a = jnp.max(jnp.where(a < m1, a, _INT_MIN), axis=1, keepdims=True)
        m2 = jnp.maximum(jnp.max(b, axis=1, keepdims=True), sec_a)
        ms += [m1, m2]
        m_prev = m2
    vals = []
    idxs = []
    for m in ms[:kp1]:
        idxs.append(_LOW_MASK - (m & _LOW_MASK))
        vals.append(_ikey_to_f32(m & _HIGH_MASK))

    row_ids = row0 + jax.lax.broadcasted_iota(jnp.int32, (block_rows, 1), 0)

    valid = [i != row_ids for i in idxs]
    mmax = functools.reduce(
        jnp.maximum,
        [jnp.where(v, x, _NEG_INF) for v, x in zip(valid, vals)])
    exps = [jnp.where(v, jnp.exp(x - mmax), 0.0)
            for v, x in zip(valid, vals)]
    denom = functools.reduce(jnp.add, exps)
    weights = [e / denom for e in exps]

    wdense = jnp.zeros((block_rows, n), jnp.float32)
    for w, idx in zip(weights, idxs):
        wdense = wdense + jnp.where(col_iota == idx, w, 0.0)

    prop = jnp.dot(wdense, proj_ref[...],
                   preferred_element_type=jnp.float32,
                   precision=jax.lax.Precision.DEFAULT)
    out_ref[...] = feat_ref[...] + _STRENGTH * (prop + emb_ref[...])


def kernel(feats, W, emb):
    feat = feats[0]
    n, d = feat.shape
    k = min(10, n // 10)
    kp1 = k + 1

    block_rows = 400 if n % 400 == 0 else 200

    proj, normed_hi, normed_lo = pl.pallas_call(
        _proj_norm_kernel,
        grid=(n // block_rows,),
        in_specs=[
            pl.BlockSpec((block_rows, d), lambda i: (i, 0)),
            pl.BlockSpec((d, d), lambda i: (0, 0)),
        ],
        out_specs=[
            pl.BlockSpec((block_rows, d), lambda i: (i, 0)),
            pl.BlockSpec((block_rows, d), lambda i: (i, 0)),
            pl.BlockSpec((block_rows, d), lambda i: (i, 0)),
        ],
        out_shape=[
            jax.ShapeDtypeStruct((n, d), jnp.float32),
            jax.ShapeDtypeStruct((n, d), jnp.bfloat16),
            jax.ShapeDtypeStruct((n, d), jnp.bfloat16),
        ],
    )(feat, W.T)

    npk = -(-n // 1280) * 1280
    out = pl.pallas_call(
        functools.partial(_topk_prop_kernel,
                          block_rows=block_rows, n=n, npk=npk, kp1=kp1),
        grid=(n // block_rows,),
        in_specs=[
            pl.BlockSpec((block_rows, d), lambda i: (i, 0)),
            pl.BlockSpec((block_rows, d), lambda i: (i, 0)),
            pl.BlockSpec((d, n), lambda i: (0, 0)),
            pl.BlockSpec((d, n), lambda i: (0, 0)),
            pl.BlockSpec((n, d), lambda i: (0, 0)),
            pl.BlockSpec((block_rows, d), lambda i: (i, 0)),
            pl.BlockSpec((1, d), lambda i: (0, 0)),
        ],
        out_specs=pl.BlockSpec((block_rows, d), lambda i: (i, 0)),
        out_shape=jax.ShapeDtypeStruct((n, d), jnp.float32),
        scratch_shapes=[pltpu.VMEM((block_rows, npk), jnp.int32)],
    )(normed_hi, normed_lo, normed_hi.T, normed_lo.T, proj, feat, emb)

    return out[None]


# speedup vs baseline: 2.4383x; 1.0299x over previous
import functools

import jax
import jax.numpy as jnp
import numpy as np
from jax.experimental import pallas as pl
from jax.experimental.pallas import tpu as pltpu

_STRENGTH = 0.1
_NEG_INF = float("-inf")
_INT_MIN = np.int32(-(2 ** 31))
_INT_MAX = np.int32(2 ** 31 - 1)
_LOW_MASK = np.int32(16383)
_HIGH_MASK = np.int32(-16384)


def _proj_norm_kernel(feat_ref, wt_ref, proj_ref, hi_ref, lo_ref):
    proj = jnp.dot(feat_ref[...], wt_ref[...],
                   preferred_element_type=jnp.float32,
                   precision=jax.lax.Precision.HIGHEST)
    proj_ref[...] = proj
    norm = jnp.sqrt(jnp.sum(proj * proj, axis=1, keepdims=True))
    normed = proj / jnp.maximum(norm, 1e-12)
    hi = normed.astype(jnp.bfloat16)
    hi_ref[...] = hi
    lo_ref[...] = (normed - hi.astype(jnp.float32)).astype(jnp.bfloat16)


def _f32_to_ikey(x):
    bits = jax.lax.bitcast_convert_type(x, jnp.int32)
    return bits ^ (jax.lax.shift_right_arithmetic(bits, 31) & _INT_MAX)


def _ikey_to_f32(k):
    bits = k ^ (jax.lax.shift_right_arithmetic(k, 31) & _INT_MAX)
    return jax.lax.bitcast_convert_type(bits, jnp.float32)


def _topk_prop_kernel(rhi_ref, rlo_ref, thi_ref, tlo_ref, proj_ref, feat_ref,
                      emb_ref, out_ref, keys_ref, *, block_rows, n, npk, kp1):
    pid = pl.program_id(0)
    row0 = pid * block_rows

    dot = functools.partial(jnp.dot, preferred_element_type=jnp.float32,
                            precision=jax.lax.Precision.DEFAULT)
    sims = (dot(rhi_ref[...], thi_ref[...])
            + dot(rhi_ref[...], tlo_ref[...])
            + dot(rlo_ref[...], thi_ref[...]))
    col_iota = jax.lax.broadcasted_iota(jnp.int32, (block_rows, n), 1)
    keys = (_f32_to_ikey(sims) & _HIGH_MASK) | (_LOW_MASK - col_iota)
    if npk > n:
        keys = jnp.concatenate(
            [keys, jnp.full((block_rows, npk - n), _INT_MIN, jnp.int32)],
            axis=1)
    keys_ref[...] = keys

    vals = []
    idxs = []
    m_prev = jnp.full((block_rows, 1), _INT_MAX, jnp.int32)
    for _ in range(kp1):
        cand = jnp.where(keys_ref[...] < m_prev, keys_ref[...], _INT_MIN)
        m = jnp.max(cand, axis=1, keepdims=True)
        idxs.append(_LOW_MASK - (m & _LOW_MASK))
        vals.append(_ikey_to_f32(m & _HIGH_MASK))
        m_prev = m

    row_ids = row0 + jax.lax.broadcasted_iota(jnp.int32, (block_rows, 1), 0)

    valid = [i != row_ids for i in idxs]
    mmax = functools.reduce(
        jnp.maximum,
        [jnp.where(v, x, _NEG_INF) for v, x in zip(valid, vals)])
    exps = [jnp.where(v, jnp.exp(x - mmax), 0.0)
            for v, x in zip(valid, vals)]
    denom = functools.reduce(jnp.add, exps)
    weights = [e / denom for e in exps]

    wdense = jnp.zeros((block_rows, n), jnp.float32)
    for w, idx in zip(weights, idxs):
        wdense = wdense + jnp.where(col_iota == idx, w, 0.0)

    prop = jnp.dot(wdense, proj_ref[...],
                   preferred_element_type=jnp.float32,
                   precision=jax.lax.Precision.DEFAULT)
    out_ref[...] = feat_ref[...] + _STRENGTH * (prop + emb_ref[...])


def kernel(feats, W, emb):
    feat = feats[0]
    n, d = feat.shape
    k = min(10, n // 10)
    kp1 = k + 1

    block_rows = 400 if n % 400 == 0 else 200

    proj, normed_hi, normed_lo = pl.pallas_call(
        _proj_norm_kernel,
        grid=(n // block_rows,),
        in_specs=[
            pl.BlockSpec((block_rows, d), lambda i: (i, 0)),
            pl.BlockSpec((d, d), lambda i: (0, 0)),
        ],
        out_specs=[
            pl.BlockSpec((block_rows, d), lambda i: (i, 0)),
            pl.BlockSpec((block_rows, d), lambda i: (i, 0)),
            pl.BlockSpec((block_rows, d), lambda i: (i, 0)),
        ],
        out_shape=[
            jax.ShapeDtypeStruct((n, d), jnp.float32),
            jax.ShapeDtypeStruct((n, d), jnp.bfloat16),
            jax.ShapeDtypeStruct((n, d), jnp.bfloat16),
        ],
    )(feat, W.T)

    npk = -(-n // 1280) * 1280
    out = pl.pallas_call(
        functools.partial(_topk_prop_kernel,
                          block_rows=block_rows, n=n, npk=npk, kp1=kp1),
        grid=(n // block_rows,),
        in_specs=[
            pl.BlockSpec((block_rows, d), lambda i: (i, 0)),
            pl.BlockSpec((block_rows, d), lambda i: (i, 0)),
            pl.BlockSpec((d, n), lambda i: (0, 0)),
            pl.BlockSpec((d, n), lambda i: (0, 0)),
            pl.BlockSpec((n, d), lambda i: (0, 0)),
            pl.BlockSpec((block_rows, d), lambda i: (i, 0)),
            pl.BlockSpec((1, d), lambda i: (0, 0)),
        ],
        out_specs=pl.BlockSpec((block_rows, d), lambda i: (i, 0)),
        out_shape=jax.ShapeDtypeStruct((n, d), jnp.float32),
        scratch_shapes=[pltpu.VMEM((block_rows, npk), jnp.int32)],
    )(normed_hi, normed_lo, normed_hi.T, normed_lo.T, proj, feat, emb)

    return out[None]
